# baseline (speedup 1.0000x reference)
import jax
import jax.numpy as jnp
from jax import lax
from jax.experimental import pallas as pl
from jax.experimental.pallas import tpu as pltpu

_N_CLASSES = 100000
_K = 5
_BATCH = 128
_BC = 25000
_S = 40
_NSL = _BC // _S
_GRID = _N_CLASSES // _BC
_NEG = float(-jnp.inf)


def _stream_body(xt_ref, y_ref, loss_ref, tie_ref, a_ref, es_ref, fy_ref):
    k = pl.program_id(0)

    @pl.when(k == 0)
    def _init():
        a_ref[...] = jnp.full_like(a_ref, _NEG)
        es_ref[...] = jnp.zeros_like(es_ref)
        fy_ref[...] = jnp.zeros_like(fy_ref)

    yv = y_ref[...]
    iota_s = lax.broadcasted_iota(jnp.int32, (_S, _BATCH), 0)
    yvb = yv + jnp.zeros((_S, _BATCH), jnp.int32)
    base0 = k * _BC

    def slab(j, c):
        a1, a2, a3, a4, es, fy = c
        t = xt_ref[pl.ds(j * _S, _S), :]
        es = es + jnp.exp(t)
        fy = fy + jnp.where(iota_s + (base0 + j * _S) == yvb, t, 0.0)
        lo = jnp.minimum(a1, t)
        a1 = jnp.maximum(a1, t)
        lo2 = jnp.minimum(a2, lo)
        a2 = jnp.maximum(a2, lo)
        lo3 = jnp.minimum(a3, lo2)
        a3 = jnp.maximum(a3, lo2)
        a4 = jnp.maximum(a4, lo3)
        return a1, a2, a3, a4, es, fy

    carry = (a_ref[0], a_ref[1], a_ref[2], a_ref[3],
             es_ref[...], fy_ref[...])
    carry = lax.fori_loop(0, _NSL, slab, carry, unroll=10)
    for i in range(4):
        a_ref[i] = carry[i]
    es_ref[...] = carry[4]
    fy_ref[...] = carry[5]

    @pl.when(k == _GRID - 1)
    def _finish():
        accs = [a_ref[i] for i in range(4)]
        es_l = jnp.sum(es_ref[...], axis=0, keepdims=True)
        fy_l = jnp.sum(fy_ref[...], axis=0, keepdims=True)

        neg_inf = jnp.float32(_NEG)
        m = accs[0][0:1, :]
        for a in accs:
            m = jnp.maximum(m, jnp.max(a, axis=0, keepdims=True))
        tops = jnp.zeros_like(fy_l)
        remaining = jnp.full_like(fy_l, float(_K))
        v5 = m
        for t_i in range(_K):
            c = jnp.zeros_like(fy_l)
            for a in accs:
                c += jnp.sum(jnp.where(a == m, 1.0, 0.0), axis=0,
                             keepdims=True)
            take = jnp.minimum(remaining, c)
            tops += take * jnp.exp(jnp.minimum(m - fy_l, 80.0))
            remaining -= take
            v5 = jnp.where(take > 0.0, m, v5)
            if t_i < _K - 1:
                nm = jnp.full_like(fy_l, _NEG)
                for a in accs:
                    nm = jnp.maximum(
                        nm, jnp.max(jnp.where(a < m, a, neg_inf),
                                    axis=0, keepdims=True))
                m = nm

        ind = jnp.where(fy_l < v5, 1.0, 0.0)
        tail = jnp.exp(-fy_l) * es_l - tops - ind
        losses = jnp.log(1.0 + tail)
        loss_ref[...] = (jnp.sum(losses) * (1.0 / _BATCH)).reshape(1, 1)
        tie = jnp.any(fy_l == v5) | jnp.any(accs[3] >= v5)
        tie_ref[...] = jnp.where(tie, 1.0, 0.0).reshape(1, 1)


def _exact_body(x_ref, y_ref, out_ref, losses_ref):
    _BR = 8
    yv = y_ref[pl.ds(pl.program_id(0) * _BR, _BR), :]
    rows = x_ref[...]
    col = lax.broadcasted_iota(jnp.int32, rows.shape, 1)
    neg_inf = jnp.float32(_NEG)

    fy = jnp.sum(jnp.where(col == yv, rows, 0.0), axis=1, keepdims=True)
    e_sum = jnp.sum(jnp.exp(rows), axis=1, keepdims=True)
    rem5 = jnp.full_like(fy, float(_K))
    tops = jnp.zeros_like(fy)
    thr = jnp.full_like(fy, jnp.inf)
    for _ in range(_K):
        mx = jnp.max(jnp.where(rows < thr, rows, neg_inf),
                     axis=1, keepdims=True)
        cx = jnp.sum(jnp.where(rows == mx, 1.0, 0.0), axis=1, keepdims=True)
        take = jnp.minimum(rem5, cx)
        tops += take * jnp.exp(jnp.minimum(mx - fy, 80.0))
        rem5 -= take
        thr = mx
    cnt_gt = jnp.sum(jnp.where(rows > fy, 1.0, 0.0), axis=1, keepdims=True)
    cnt_eqb = jnp.sum(jnp.where((rows == fy) & (col < yv), 1.0, 0.0),
                      axis=1, keepdims=True)
    ind = jnp.where(cnt_gt + cnt_eqb >= float(_K), 1.0, 0.0)
    tail = jnp.exp(-fy) * e_sum - tops - ind
    losses_ref[...] = jnp.log(1.0 + tail)

    @pl.when(pl.program_id(0) == 0)
    def _init():
        out_ref[...] = jnp.zeros_like(out_ref)

    out_ref[...] += jnp.sum(losses_ref[...]).reshape(1, 1) * (1.0 / _BATCH)


def _exact_loss(x, y2):
    return pl.pallas_call(
        _exact_body,
        grid=(16,),
        in_specs=[
            pl.BlockSpec((8, _N_CLASSES), lambda i: (i, 0)),
            pl.BlockSpec((_BATCH, 1), lambda i: (0, 0)),
        ],
        out_specs=pl.BlockSpec((1, 1), lambda i: (0, 0)),
        out_shape=jax.ShapeDtypeStruct((1, 1), jnp.float32),
        scratch_shapes=[pltpu.VMEM((8, 1), jnp.float32)],
    )(x, y2)[0, 0]


@jax.jit
def kernel(x, y):
    xt = x.T
    yr = y.reshape(1, _BATCH)
    loss, tie = pl.pallas_call(
        _stream_body,
        grid=(_GRID,),
        in_specs=[
            pl.BlockSpec((_BC, _BATCH), lambda i: (i, 0)),
            pl.BlockSpec((1, _BATCH), lambda i: (0, 0)),
        ],
        out_specs=[
            pl.BlockSpec((1, 1), lambda i: (0, 0)),
            pl.BlockSpec((1, 1), lambda i: (0, 0)),
        ],
        out_shape=[
            jax.ShapeDtypeStruct((1, 1), jnp.float32),
            jax.ShapeDtypeStruct((1, 1), jnp.float32),
        ],
        scratch_shapes=[
            pltpu.VMEM((4, _S, _BATCH), jnp.float32),
            pltpu.VMEM((_S, _BATCH), jnp.float32),
            pltpu.VMEM((_S, _BATCH), jnp.float32),
        ],
    )(xt, yr)
    y2 = y.reshape(_BATCH, 1)
    return lax.cond(tie[0, 0] > 0.0,
                    lambda ops: _exact_loss(*ops),
                    lambda ops: loss[0, 0],
                    (x, y2))


# speedup vs baseline: 1.0276x; 1.0276x over previous
import jax
import jax.numpy as jnp
from jax import lax
from jax.experimental import pallas as pl
from jax.experimental.pallas import tpu as pltpu

_N_CLASSES = 100000
_K = 5
_BATCH = 128
_BC = 20000
_S = 40
_NSL = _BC // _S
_GRID = _N_CLASSES // _BC
_NEG = float(-jnp.inf)


def _stream_body(xt_ref, y_ref, loss_ref, tie_ref, a_ref, es_ref, fy_ref):
    k = pl.program_id(0)

    @pl.when(k == 0)
    def _init():
        a_ref[...] = jnp.full_like(a_ref, _NEG)
        es_ref[...] = jnp.zeros_like(es_ref)
        fy_ref[...] = jnp.zeros_like(fy_ref)

    yv = y_ref[...]
    iota_s = lax.broadcasted_iota(jnp.int32, (_S, _BATCH), 0)
    yvb = yv + jnp.zeros((_S, _BATCH), jnp.int32)
    base0 = k * _BC

    def slab(j, c):
        a1, a2, a3, a4, es, fy = c
        t = xt_ref[pl.ds(j * _S, _S), :]
        es = es + jnp.exp(t)
        fy = fy + jnp.where(iota_s + (base0 + j * _S) == yvb, t, 0.0)
        lo = jnp.minimum(a1, t)
        a1 = jnp.maximum(a1, t)
        lo2 = jnp.minimum(a2, lo)
        a2 = jnp.maximum(a2, lo)
        lo3 = jnp.minimum(a3, lo2)
        a3 = jnp.maximum(a3, lo2)
        a4 = jnp.maximum(a4, lo3)
        return a1, a2, a3, a4, es, fy

    carry = (a_ref[0], a_ref[1], a_ref[2], a_ref[3],
             es_ref[...], fy_ref[...])
    carry = lax.fori_loop(0, _NSL, slab, carry, unroll=10)
    for i in range(4):
        a_ref[i] = carry[i]
    es_ref[...] = carry[4]
    fy_ref[...] = carry[5]

    @pl.when(k == _GRID - 1)
    def _finish():
        accs = [a_ref[i] for i in range(4)]
        es_l = jnp.sum(es_ref[...], axis=0, keepdims=True)
        fy_l = jnp.sum(fy_ref[...], axis=0, keepdims=True)

        neg_inf = jnp.float32(_NEG)
        m = accs[0][0:1, :]
        for a in accs:
            m = jnp.maximum(m, jnp.max(a, axis=0, keepdims=True))
        tops = jnp.zeros_like(fy_l)
        remaining = jnp.full_like(fy_l, float(_K))
        v5 = m
        for t_i in range(_K):
            c = jnp.zeros_like(fy_l)
            for a in accs:
                c += jnp.sum(jnp.where(a == m, 1.0, 0.0), axis=0,
                             keepdims=True)
            take = jnp.minimum(remaining, c)
            tops += take * jnp.exp(jnp.minimum(m - fy_l, 80.0))
            remaining -= take
            v5 = jnp.where(take > 0.0, m, v5)
            if t_i < _K - 1:
                nm = jnp.full_like(fy_l, _NEG)
                for a in accs:
                    nm = jnp.maximum(
                        nm, jnp.max(jnp.where(a < m, a, neg_inf),
                                    axis=0, keepdims=True))
                m = nm

        ind = jnp.where(fy_l < v5, 1.0, 0.0)
        tail = jnp.exp(-fy_l) * es_l - tops - ind
        losses = jnp.log(1.0 + tail)
        loss_ref[...] = (jnp.sum(losses) * (1.0 / _BATCH)).reshape(1, 1)
        tie = jnp.any(fy_l == v5) | jnp.any(accs[3] >= v5)
        tie_ref[...] = jnp.where(tie, 1.0, 0.0).reshape(1, 1)


def _exact_body(x_ref, y_ref, out_ref, losses_ref):
    _BR = 8
    yv = y_ref[pl.ds(pl.program_id(0) * _BR, _BR), :]
    rows = x_ref[...]
    col = lax.broadcasted_iota(jnp.int32, rows.shape, 1)
    neg_inf = jnp.float32(_NEG)

    fy = jnp.sum(jnp.where(col == yv, rows, 0.0), axis=1, keepdims=True)
    e_sum = jnp.sum(jnp.exp(rows), axis=1, keepdims=True)
    rem5 = jnp.full_like(fy, float(_K))
    tops = jnp.zeros_like(fy)
    thr = jnp.full_like(fy, jnp.inf)
    for _ in range(_K):
        mx = jnp.max(jnp.where(rows < thr, rows, neg_inf),
                     axis=1, keepdims=True)
        cx = jnp.sum(jnp.where(rows == mx, 1.0, 0.0), axis=1, keepdims=True)
        take = jnp.minimum(rem5, cx)
        tops += take * jnp.exp(jnp.minimum(mx - fy, 80.0))
        rem5 -= take
        thr = mx
    cnt_gt = jnp.sum(jnp.where(rows > fy, 1.0, 0.0), axis=1, keepdims=True)
    cnt_eqb = jnp.sum(jnp.where((rows == fy) & (col < yv), 1.0, 0.0),
                      axis=1, keepdims=True)
    ind = jnp.where(cnt_gt + cnt_eqb >= float(_K), 1.0, 0.0)
    tail = jnp.exp(-fy) * e_sum - tops - ind
    losses_ref[...] = jnp.log(1.0 + tail)

    @pl.when(pl.program_id(0) == 0)
    def _init():
        out_ref[...] = jnp.zeros_like(out_ref)

    out_ref[...] += jnp.sum(losses_ref[...]).reshape(1, 1) * (1.0 / _BATCH)


def _exact_loss(x, y2):
    return pl.pallas_call(
        _exact_body,
        grid=(16,),
        in_specs=[
            pl.BlockSpec((8, _N_CLASSES), lambda i: (i, 0)),
            pl.BlockSpec((_BATCH, 1), lambda i: (0, 0)),
        ],
        out_specs=pl.BlockSpec((1, 1), lambda i: (0, 0)),
        out_shape=jax.ShapeDtypeStruct((1, 1), jnp.float32),
        scratch_shapes=[pltpu.VMEM((8, 1), jnp.float32)],
    )(x, y2)[0, 0]


@jax.jit
def kernel(x, y):
    xt = x.T
    yr = y.reshape(1, _BATCH)
    loss, tie = pl.pallas_call(
        _stream_body,
        grid=(_GRID,),
        in_specs=[
            pl.BlockSpec((_BC, _BATCH), lambda i: (i, 0)),
            pl.BlockSpec((1, _BATCH), lambda i: (0, 0)),
        ],
        out_specs=[
            pl.BlockSpec((1, 1), lambda i: (0, 0)),
            pl.BlockSpec((1, 1), lambda i: (0, 0)),
        ],
        out_shape=[
            jax.ShapeDtypeStruct((1, 1), jnp.float32),
            jax.ShapeDtypeStruct((1, 1), jnp.float32),
        ],
        scratch_shapes=[
            pltpu.VMEM((4, _S, _BATCH), jnp.float32),
            pltpu.VMEM((_S, _BATCH), jnp.float32),
            pltpu.VMEM((_S, _BATCH), jnp.float32),
        ],
    )(xt, yr)
    y2 = y.reshape(_BATCH, 1)
    return lax.cond(tie[0, 0] > 0.0,
                    lambda ops: _exact_loss(*ops),
                    lambda ops: loss[0, 0],
                    (x, y2))
